# baseline (speedup 1.0000x reference)
import functools

import jax
import jax.numpy as jnp
from jax import lax
from jax.experimental import pallas as pl
from jax.experimental.pallas import tpu as pltpu
from jax.experimental.pallas import tpu_sc as plsc

BATCH = 16384
DIM = 128
NROWS = 8
CHUNK = 128


def _body(types_hbm, labels_hbm, table_hbm, out_hbm,
          t_v, l_v, table_v, stab, rows_v, gsem, osem, *, bpw):
    info = plsc.get_sparse_core_info()
    nc, lanes = info.num_cores, info.num_lanes
    nchunk = bpw // CHUNK

    sid = lax.axis_index("s")
    wid = sid * nc + lax.axis_index("c")
    base = wid * bpw
    row_off = sid * NROWS

    pltpu.sync_copy(table_hbm, table_v)
    pltpu.sync_copy(table_v, stab.at[pl.ds(row_off, NROWS)])

    pltpu.sync_copy(types_hbm.at[pl.ds(base, bpw)], t_v)
    pltpu.sync_copy(labels_hbm.at[pl.ds(base, bpw)], l_v)

    gpc = CHUNK // lanes

    for j in range(nchunk):
        def fire(g, carry, j=j):
            off = pl.multiple_of(g * lanes, lanes)
            s = pl.ds(off, lanes)
            ids = t_v[s] * 2 + l_v[s] + row_off
            pltpu.async_copy(stab.at[ids], rows_v.at[s], gsem.at[j])
            return carry
        lax.fori_loop(j * gpc, (j + 1) * gpc, fire, 0, unroll=False)

    stores = []
    for j in range(nchunk):
        rows = pl.ds(j * CHUNK, CHUNK)
        pltpu.make_async_copy(out_hbm.at[pl.ds(base + j * CHUNK, CHUNK)],
                              rows_v.at[rows], gsem.at[j]).wait()
        stores.append(
            pltpu.async_copy(rows_v.at[rows],
                             out_hbm.at[pl.ds(base + j * CHUNK, CHUNK)],
                             osem))
    for s_ in stores:
        s_.wait()


def _sc_call(types, labels, table):
    info = plsc.get_sparse_core_info()
    nw = info.num_cores * info.num_subcores
    n = types.shape[0]
    bpw = n // nw
    nchunk = bpw // CHUNK
    mesh = plsc.VectorSubcoreMesh(core_axis_name="c", subcore_axis_name="s")
    f = functools.partial(
        pl.kernel,
        mesh=mesh,
        out_type=jax.ShapeDtypeStruct((n, DIM), jnp.float32),
        scratch_types=[
            pltpu.VMEM((bpw,), jnp.int32),
            pltpu.VMEM((bpw,), jnp.int32),
            pltpu.VMEM((NROWS, DIM), jnp.float32),
            pltpu.VMEM_SHARED((16 * NROWS, DIM), jnp.float32),
            pltpu.VMEM((bpw, DIM), jnp.float32),
            pltpu.SemaphoreType.DMA((nchunk,)),
            pltpu.SemaphoreType.DMA,
        ],
    )(functools.partial(_body, bpw=bpw))
    return f(types, labels, table)


def kernel(interaction_types, labels, embedding_weight):
    return _sc_call(interaction_types.astype(jnp.int32),
                    labels.astype(jnp.int32),
                    embedding_weight)


# speedup vs baseline: 1.0049x; 1.0049x over previous
import functools

import jax
import jax.numpy as jnp
from jax import lax
from jax.experimental import pallas as pl
from jax.experimental.pallas import tpu as pltpu
from jax.experimental.pallas import tpu_sc as plsc

BATCH = 16384
DIM = 128
NROWS = 8
CHUNK = 128


def _body(types_hbm, labels_hbm, table_hbm, out_hbm,
          t_v, l_v, stab, rows_v, gsem, osem, *, bpw):
    info = plsc.get_sparse_core_info()
    nc, lanes = info.num_cores, info.num_lanes
    nchunk = bpw // CHUNK

    sid = lax.axis_index("s")
    wid = sid * nc + lax.axis_index("c")
    base = wid * bpw
    row_off = sid * NROWS

    pltpu.sync_copy(table_hbm, stab.at[pl.ds(row_off, NROWS)])

    pltpu.sync_copy(types_hbm.at[pl.ds(base, bpw)], t_v)
    pltpu.sync_copy(labels_hbm.at[pl.ds(base, bpw)], l_v)

    gpc = CHUNK // lanes

    for j in range(nchunk):
        def fire(g, carry, j=j):
            off = pl.multiple_of(g * lanes, lanes)
            s = pl.ds(off, lanes)
            ids = t_v[s] * 2 + l_v[s] + row_off
            pltpu.async_copy(stab.at[ids], rows_v.at[s], gsem.at[j])
            return carry
        lax.fori_loop(j * gpc, (j + 1) * gpc, fire, 0, unroll=False)

    stores = []
    for j in range(nchunk):
        rows = pl.ds(j * CHUNK, CHUNK)
        pltpu.make_async_copy(out_hbm.at[pl.ds(base + j * CHUNK, CHUNK)],
                              rows_v.at[rows], gsem.at[j]).wait()
        stores.append(
            pltpu.async_copy(rows_v.at[rows],
                             out_hbm.at[pl.ds(base + j * CHUNK, CHUNK)],
                             osem))
    for s_ in stores:
        s_.wait()


def _sc_call(types, labels, table):
    info = plsc.get_sparse_core_info()
    nw = info.num_cores * info.num_subcores
    n = types.shape[0]
    bpw = n // nw
    nchunk = bpw // CHUNK
    mesh = plsc.VectorSubcoreMesh(core_axis_name="c", subcore_axis_name="s")
    f = functools.partial(
        pl.kernel,
        mesh=mesh,
        out_type=jax.ShapeDtypeStruct((n, DIM), jnp.float32),
        scratch_types=[
            pltpu.VMEM((bpw,), jnp.int32),
            pltpu.VMEM((bpw,), jnp.int32),
            pltpu.VMEM_SHARED((16 * NROWS, DIM), jnp.float32),
            pltpu.VMEM((bpw, DIM), jnp.float32),
            pltpu.SemaphoreType.DMA((nchunk,)),
            pltpu.SemaphoreType.DMA,
        ],
    )(functools.partial(_body, bpw=bpw))
    return f(types, labels, table)


def kernel(interaction_types, labels, embedding_weight):
    return _sc_call(interaction_types.astype(jnp.int32),
                    labels.astype(jnp.int32),
                    embedding_weight)


# speedup vs baseline: 1.0150x; 1.0101x over previous
import functools

import jax
import jax.numpy as jnp
from jax import lax
from jax.experimental import pallas as pl
from jax.experimental.pallas import tpu as pltpu
from jax.experimental.pallas import tpu_sc as plsc

BATCH = 16384
DIM = 128
NROWS = 8
CHUNK = 64


def _body(types_hbm, labels_hbm, table_hbm, out_hbm,
          t_v, l_v, stab, rows_v, gsem, osem, *, bpw):
    info = plsc.get_sparse_core_info()
    nc, lanes = info.num_cores, info.num_lanes
    nchunk = bpw // CHUNK

    sid = lax.axis_index("s")
    wid = sid * nc + lax.axis_index("c")
    base = wid * bpw
    row_off = sid * NROWS

    pltpu.sync_copy(table_hbm, stab.at[pl.ds(row_off, NROWS)])

    pltpu.sync_copy(types_hbm.at[pl.ds(base, bpw)], t_v)
    pltpu.sync_copy(labels_hbm.at[pl.ds(base, bpw)], l_v)

    gpc = CHUNK // lanes

    for j in range(nchunk):
        def fire(g, carry, j=j):
            off = pl.multiple_of(g * lanes, lanes)
            s = pl.ds(off, lanes)
            ids = t_v[s] * 2 + l_v[s] + row_off
            pltpu.async_copy(stab.at[ids], rows_v.at[s], gsem.at[j])
            return carry
        lax.fori_loop(j * gpc, (j + 1) * gpc, fire, 0, unroll=False)

    stores = []
    for j in range(nchunk):
        rows = pl.ds(j * CHUNK, CHUNK)
        pltpu.make_async_copy(out_hbm.at[pl.ds(base + j * CHUNK, CHUNK)],
                              rows_v.at[rows], gsem.at[j]).wait()
        stores.append(
            pltpu.async_copy(rows_v.at[rows],
                             out_hbm.at[pl.ds(base + j * CHUNK, CHUNK)],
                             osem))
    for s_ in stores:
        s_.wait()


def _sc_call(types, labels, table):
    info = plsc.get_sparse_core_info()
    nw = info.num_cores * info.num_subcores
    n = types.shape[0]
    bpw = n // nw
    nchunk = bpw // CHUNK
    mesh = plsc.VectorSubcoreMesh(core_axis_name="c", subcore_axis_name="s")
    f = functools.partial(
        pl.kernel,
        mesh=mesh,
        out_type=jax.ShapeDtypeStruct((n, DIM), jnp.float32),
        scratch_types=[
            pltpu.VMEM((bpw,), jnp.int32),
            pltpu.VMEM((bpw,), jnp.int32),
            pltpu.VMEM_SHARED((16 * NROWS, DIM), jnp.float32),
            pltpu.VMEM((bpw, DIM), jnp.float32),
            pltpu.SemaphoreType.DMA((nchunk,)),
            pltpu.SemaphoreType.DMA,
        ],
    )(functools.partial(_body, bpw=bpw))
    return f(types, labels, table)


def kernel(interaction_types, labels, embedding_weight):
    return _sc_call(interaction_types.astype(jnp.int32),
                    labels.astype(jnp.int32),
                    embedding_weight)
